# baseline (speedup 1.0000x reference)
import jax
import jax.numpy as jnp
from jax import lax
from jax.experimental import pallas as pl
from jax.experimental.pallas import tpu as pltpu
from jax.experimental.pallas import tpu_sc as plsc

_NC = 2
_NS = 16
_NW = _NC * _NS
_CH = 25
_LANES = 16


def _sc_body(xp_hbm, xt_hbm, sc_hbm, acc_out, cnt_out, xp_v, xt_v, sc_v, res_v):
    n_chunks = xp_hbm.shape[0] // _CH
    per_w = n_chunks // _NW
    extra = n_chunks - per_w * _NW

    wid = lax.axis_index("s") * _NC + lax.axis_index("c")
    start = wid * per_w + jnp.minimum(wid, extra)
    cnt_chunks = per_w + jnp.where(wid < extra, 1, 0)

    zero = jnp.zeros((_LANES,), jnp.float32)

    def chunk_body(ci, carry):
        acc, cnt = carry
        t0 = (start + ci) * _CH
        pltpu.sync_copy(xp_hbm.at[pl.ds(t0, _CH)], xp_v)
        pltpu.sync_copy(xt_hbm.at[pl.ds(t0, _CH)], xt_v)
        pltpu.sync_copy(sc_hbm.at[pl.ds(t0 * 128, _CH * 128)], sc_v)

        def tile_body(t, carry2):
            acc2, cnt2 = carry2
            for j in range(128 // _LANES):
                mv = sc_v[pl.ds(t * 128 + j * _LANES, _LANES)]
                msk = mv == 1
                cnt2 = cnt2 + jnp.where(msk, 1.0, 0.0)
                for c in range(4):
                    d = xp_v[t, c, pl.ds(j * _LANES, _LANES)] - xt_v[t, c, pl.ds(j * _LANES, _LANES)]
                    ad = jnp.abs(d)
                    mn = jnp.minimum(ad, 1.0)
                    pe = mn * (ad - 0.5 * mn)
                    acc2 = acc2 + jnp.where(msk, pe, 0.0)
            return acc2, cnt2

        return lax.fori_loop(0, _CH, tile_body, (acc, cnt))

    acc, cnt = lax.fori_loop(0, cnt_chunks, chunk_body, (zero, zero))
    res_v[0, pl.ds(0, _LANES)] = acc
    res_v[1, pl.ds(0, _LANES)] = cnt
    pltpu.sync_copy(res_v.at[0], acc_out.at[wid])
    pltpu.sync_copy(res_v.at[1], cnt_out.at[wid])


@jax.jit
def _run(y_pred, y_true, score):
    n = y_pred.shape[0]
    t = n // 128

    def as_tiles(x):
        return x.reshape(t, 128, 4).transpose(0, 2, 1)

    xp = as_tiles(y_pred)
    xt = as_tiles(y_true)
    sc = score

    acc, cnt = pl.kernel(
        _sc_body,
        out_type=(
            jax.ShapeDtypeStruct((_NW, _LANES), jnp.float32),
            jax.ShapeDtypeStruct((_NW, _LANES), jnp.float32),
        ),
        mesh=plsc.VectorSubcoreMesh(core_axis_name="c", subcore_axis_name="s"),
        scratch_types=[
            pltpu.VMEM((_CH, 4, 128), jnp.float32),
            pltpu.VMEM((_CH, 4, 128), jnp.float32),
            pltpu.VMEM((_CH * 128,), jnp.int32),
            pltpu.VMEM((2, _LANES), jnp.float32),
        ],
    )(xp, xt, sc)
    return jnp.sum(acc) / (4.0 * jnp.sum(cnt))


def kernel(y_pred, y_true, y_true_score):
    return _run(y_pred, y_true, y_true_score.astype(jnp.int32))


# speedup vs baseline: 1.6640x; 1.6640x over previous
import jax
import jax.numpy as jnp
from jax import lax
from jax.experimental import pallas as pl
from jax.experimental.pallas import tpu as pltpu
from jax.experimental.pallas import tpu_sc as plsc

_NC = 2
_NS = 16
_NW = _NC * _NS
_CH = 50
_LANES = 16


def _sc_body(xp_hbm, xt_hbm, sc_hbm, acc_out, cnt_out,
             xp_v, xt_v, sc_v, res_v, sems):
    n_chunks = xp_hbm.shape[0] // _CH
    per_w = n_chunks // _NW
    extra = n_chunks - per_w * _NW

    wid = lax.axis_index("s") * _NC + lax.axis_index("c")
    start = wid * per_w + jnp.minimum(wid, extra)
    cnt_chunks = per_w + jnp.where(wid < extra, 1, 0)

    def dma_start(idx, b):
        t0 = (start + idx) * _CH
        pltpu.async_copy(xp_hbm.at[pl.ds(t0, _CH)], xp_v.at[b], sems.at[b])
        pltpu.async_copy(xt_hbm.at[pl.ds(t0, _CH)], xt_v.at[b], sems.at[b])
        pltpu.async_copy(sc_hbm.at[pl.ds(t0 * 128, _CH * 128)], sc_v.at[b], sems.at[b])

    def dma_wait(idx, b):
        t0 = (start + idx) * _CH
        pltpu.make_async_copy(xp_hbm.at[pl.ds(t0, _CH)], xp_v.at[b], sems.at[b]).wait()
        pltpu.make_async_copy(xt_hbm.at[pl.ds(t0, _CH)], xt_v.at[b], sems.at[b]).wait()
        pltpu.make_async_copy(sc_hbm.at[pl.ds(t0 * 128, _CH * 128)], sc_v.at[b], sems.at[b]).wait()

    def chunk_partial(b):
        def tile_body(t, carry2):
            acc2, cnt2 = carry2
            for j in range(128 // _LANES):
                mv = sc_v[b, pl.ds(t * 128 + j * _LANES, _LANES)]
                msk = mv == 1
                cnt2 = cnt2 + jnp.where(msk, 1.0, 0.0)
                for c in range(4):
                    d = (xp_v[b, t, c, pl.ds(j * _LANES, _LANES)]
                         - xt_v[b, t, c, pl.ds(j * _LANES, _LANES)])
                    ad = jnp.abs(d)
                    mn = jnp.minimum(ad, 1.0)
                    pe = mn * (ad - 0.5 * mn)
                    acc2 = acc2 + jnp.where(msk, pe, 0.0)
            return acc2, cnt2

        zero = jnp.zeros((_LANES,), jnp.float32)
        return lax.fori_loop(0, _CH, tile_body, (zero, zero))

    @pl.when(cnt_chunks > 0)
    def _p0():
        dma_start(0, 0)

    @pl.when(cnt_chunks > 1)
    def _p1():
        dma_start(1, 1)

    def group_body(g, carry):
        acc, cnt = carry
        for b in range(2):
            idx = 2 * g + b
            in_range = idx < cnt_chunks

            @pl.when(in_range)
            def _w():
                dma_wait(idx, b)

            acc_c, cnt_c = chunk_partial(b)
            acc = acc + jnp.where(in_range, acc_c, 0.0)
            cnt = cnt + jnp.where(in_range, cnt_c, 0.0)

            @pl.when(idx + 2 < cnt_chunks)
            def _s():
                dma_start(idx + 2, b)
        return acc, cnt

    zero = jnp.zeros((_LANES,), jnp.float32)
    acc, cnt = lax.fori_loop(0, (cnt_chunks + 1) // 2, group_body, (zero, zero))

    res_v[0, pl.ds(0, _LANES)] = acc
    res_v[1, pl.ds(0, _LANES)] = cnt
    pltpu.sync_copy(res_v.at[0], acc_out.at[wid])
    pltpu.sync_copy(res_v.at[1], cnt_out.at[wid])


@jax.jit
def _run(y_pred, y_true, score):
    n = y_pred.shape[0]
    t = n // 128

    def as_tiles(x):
        return x.reshape(t, 128, 4).transpose(0, 2, 1)

    xp = as_tiles(y_pred)
    xt = as_tiles(y_true)

    acc, cnt = pl.kernel(
        _sc_body,
        out_type=(
            jax.ShapeDtypeStruct((_NW, _LANES), jnp.float32),
            jax.ShapeDtypeStruct((_NW, _LANES), jnp.float32),
        ),
        mesh=plsc.VectorSubcoreMesh(core_axis_name="c", subcore_axis_name="s"),
        scratch_types=[
            pltpu.VMEM((2, _CH, 4, 128), jnp.float32),
            pltpu.VMEM((2, _CH, 4, 128), jnp.float32),
            pltpu.VMEM((2, _CH * 128), jnp.int32),
            pltpu.VMEM((2, _LANES), jnp.float32),
            pltpu.SemaphoreType.DMA((2,)),
        ],
    )(xp, xt, score)
    return jnp.sum(acc) / (4.0 * jnp.sum(cnt))


def kernel(y_pred, y_true, y_true_score):
    return _run(y_pred, y_true, y_true_score.astype(jnp.int32))


# speedup vs baseline: 1.8505x; 1.1120x over previous
import functools

import jax
import jax.numpy as jnp
from jax import lax
from jax.experimental import pallas as pl
from jax.experimental.pallas import tpu as pltpu
from jax.experimental.pallas import tpu_sc as plsc

_NC = 2
_NS = 16
_NW = _NC * _NS
_CH = 50
_LANES = 16

_T_TC = 10400
_BLOCK = 512


def _sc_body(xp_hbm, xt_hbm, sc_hbm, acc_out, cnt_out,
             xp_v, xt_v, sc_v, res_v, sems):
    n_tiles = xp_hbm.shape[0]
    n_chunks = (n_tiles - _T_TC) // _CH
    per_w = n_chunks // _NW
    extra = n_chunks - per_w * _NW

    wid = lax.axis_index("s") * _NC + lax.axis_index("c")
    start = wid * per_w + jnp.minimum(wid, extra)
    cnt_chunks = per_w + jnp.where(wid < extra, 1, 0)

    def dma_start(idx, b):
        t0 = _T_TC + (start + idx) * _CH
        pltpu.async_copy(xp_hbm.at[pl.ds(t0, _CH)], xp_v.at[b], sems.at[b])
        pltpu.async_copy(xt_hbm.at[pl.ds(t0, _CH)], xt_v.at[b], sems.at[b])
        pltpu.async_copy(sc_hbm.at[pl.ds(t0 * 128, _CH * 128)], sc_v.at[b], sems.at[b])

    def dma_wait(idx, b):
        t0 = _T_TC + (start + idx) * _CH
        pltpu.make_async_copy(xp_hbm.at[pl.ds(t0, _CH)], xp_v.at[b], sems.at[b]).wait()
        pltpu.make_async_copy(xt_hbm.at[pl.ds(t0, _CH)], xt_v.at[b], sems.at[b]).wait()
        pltpu.make_async_copy(sc_hbm.at[pl.ds(t0 * 128, _CH * 128)], sc_v.at[b], sems.at[b]).wait()

    def chunk_partial(b):
        def tile_body(t, carry2):
            acc2, cnt2 = carry2
            for j in range(128 // _LANES):
                mv = sc_v[b, pl.ds(t * 128 + j * _LANES, _LANES)]
                msk = mv == 1
                cnt2 = cnt2 + jnp.where(msk, 1.0, 0.0)
                pes = None
                for c in range(4):
                    d = (xp_v[b, t, c, pl.ds(j * _LANES, _LANES)]
                         - xt_v[b, t, c, pl.ds(j * _LANES, _LANES)])
                    ad = jnp.abs(d)
                    mn = jnp.minimum(ad, 1.0)
                    pe = mn * (ad - 0.5 * mn)
                    pes = pe if pes is None else pes + pe
                acc2 = acc2 + jnp.where(msk, pes, 0.0)
            return acc2, cnt2

        zero = jnp.zeros((_LANES,), jnp.float32)
        return lax.fori_loop(0, _CH, tile_body, (zero, zero))

    @pl.when(cnt_chunks > 0)
    def _p0():
        dma_start(0, 0)

    @pl.when(cnt_chunks > 1)
    def _p1():
        dma_start(1, 1)

    def group_body(g, carry):
        acc, cnt = carry
        for b in range(2):
            idx = 2 * g + b
            in_range = idx < cnt_chunks

            @pl.when(in_range)
            def _w():
                dma_wait(idx, b)

            acc_c, cnt_c = chunk_partial(b)
            acc = acc + jnp.where(in_range, acc_c, 0.0)
            cnt = cnt + jnp.where(in_range, cnt_c, 0.0)

            @pl.when(idx + 2 < cnt_chunks)
            def _s():
                dma_start(idx + 2, b)
        return acc, cnt

    zero = jnp.zeros((_LANES,), jnp.float32)
    acc, cnt = lax.fori_loop(0, (cnt_chunks + 1) // 2, group_body, (zero, zero))

    res_v[0, pl.ds(0, _LANES)] = acc
    res_v[1, pl.ds(0, _LANES)] = cnt
    pltpu.sync_copy(res_v.at[0], acc_out.at[wid])
    pltpu.sync_copy(res_v.at[1], cnt_out.at[wid])


def _sc_part(xp_tiles, xt_tiles, score):
    return pl.kernel(
        _sc_body,
        out_type=(
            jax.ShapeDtypeStruct((_NW, _LANES), jnp.float32),
            jax.ShapeDtypeStruct((_NW, _LANES), jnp.float32),
        ),
        mesh=plsc.VectorSubcoreMesh(core_axis_name="c", subcore_axis_name="s"),
        scratch_types=[
            pltpu.VMEM((2, _CH, 4, 128), jnp.float32),
            pltpu.VMEM((2, _CH, 4, 128), jnp.float32),
            pltpu.VMEM((2, _CH * 128), jnp.int32),
            pltpu.VMEM((2, _LANES), jnp.float32),
            pltpu.SemaphoreType.DMA((2,)),
        ],
    )(xp_tiles, xt_tiles, score)


def _tc_kernel(xp_ref, xt_ref, sc_ref, out_ref, acc_ref, cnt_ref, *, m_rows, n_blocks):
    gi = pl.program_id(0)

    @pl.when(gi == 0)
    def _init():
        acc_ref[...] = jnp.zeros_like(acc_ref)
        cnt_ref[...] = jnp.zeros_like(cnt_ref)

    d = xp_ref[...] - xt_ref[...]
    ad = jnp.abs(d)
    mn = jnp.minimum(ad, 1.0)
    pe = mn * (ad - 0.5 * mn)

    row0 = gi * _BLOCK
    valid = (jax.lax.broadcasted_iota(jnp.int32, (_BLOCK, 2, 128), 0) + row0) < m_rows
    mb = (sc_ref[...] == 1) & valid

    masked = jnp.zeros((_BLOCK, 128), jnp.float32)
    for s in range(8):
        masked += jnp.where(mb[:, s // 4, :], pe[:, s, :], 0.0)

    acc_ref[...] += jnp.sum(masked).reshape(1, 1)
    cnt_ref[...] += jnp.sum(jnp.where(mb, 1.0, 0.0)).reshape(1, 1)

    @pl.when(gi == n_blocks - 1)
    def _finish():
        out_ref[...] = jnp.concatenate([acc_ref[...], cnt_ref[...]], axis=1)


def _tc_part(xp8, xt8, sc3):
    m_rows = _T_TC // 2
    n_blocks = pl.cdiv(m_rows, _BLOCK)
    return pl.pallas_call(
        functools.partial(_tc_kernel, m_rows=m_rows, n_blocks=n_blocks),
        grid=(n_blocks,),
        in_specs=[
            pl.BlockSpec((_BLOCK, 8, 128), lambda i: (i, 0, 0)),
            pl.BlockSpec((_BLOCK, 8, 128), lambda i: (i, 0, 0)),
            pl.BlockSpec((_BLOCK, 2, 128), lambda i: (i, 0, 0)),
        ],
        out_specs=pl.BlockSpec((1, 2), lambda i: (0, 0)),
        out_shape=jax.ShapeDtypeStruct((1, 2), jnp.float32),
        scratch_shapes=[
            pltpu.VMEM((1, 1), jnp.float32),
            pltpu.VMEM((1, 1), jnp.float32),
        ],
    )(xp8, xt8, sc3)


@jax.jit
def _run(y_pred, y_true, score):
    n = y_pred.shape[0]
    t = n // 128

    def as_tiles(x):
        return x.reshape(t, 128, 4).transpose(0, 2, 1)

    def as_dense8(x):
        return x.reshape(t // 2, 2, 128, 4).transpose(0, 1, 3, 2).reshape(t // 2, 8, 128)

    sc_acc, sc_cnt = _sc_part(as_tiles(y_pred), as_tiles(y_true), score)
    tc_out = _tc_part(as_dense8(y_pred), as_dense8(y_true),
                      score.reshape(t // 2, 2, 128))

    total = tc_out[0, 0] + jnp.sum(sc_acc)
    count = tc_out[0, 1] + jnp.sum(sc_cnt)
    return total / (4.0 * count)


def kernel(y_pred, y_true, y_true_score):
    return _run(y_pred, y_true, y_true_score.astype(jnp.int32))
